# baseline (speedup 1.0000x reference)
import jax
import jax.numpy as jnp
from jax import lax
from jax.experimental import pallas as pl
from jax.experimental.pallas import tpu as pltpu
from jax.experimental.pallas import tpu_sc as plsc

_NDB = 8432
_B, _P, _G = 64, 8192, 256
_LVL_N = (5476, 2166, 600, 150, 36, 4)
_LVL_OFF = (0, 5476, 7642, 8242, 8392, 8428)
_NC, _NS, _L = 2, 16, 16
_NW = _NC * _NS
_BPW = _B // _NW


def _prep_body(db_ref, gt_ref, dbt_ref, gtt_ref):
    db = db_ref[...]
    dbt_ref[0:2, :] = db[0:2, :]
    dbt_ref[2:4, :] = jnp.log(db[2:4, :])
    gt = gt_ref[...]
    gtt_ref[:, 0:2, :] = gt[:, 0:2, :]
    gtt_ref[:, 2:4, :] = jnp.log(gt[:, 2:4, :])


def _sc_body(l0, l1, l2, l3, l4, l5, gtt_hbm, dbt_hbm, pr_hbm,
             out_hbm, flat_v, dbt_v, gtt_v, pr_v, out_v, sem):
    wid = lax.axis_index("c") * _NS + lax.axis_index("s")
    pltpu.sync_copy(dbt_hbm, dbt_v)
    cols = [jnp.full((_L,), c, jnp.int32) for c in range(4)]
    evens = lax.iota(jnp.int32, _L) * 2
    odds = evens + 1
    levels = (l0, l1, l2, l3, l4, l5)

    def batch_loss(b):
        copies = [pltpu.async_copy(lvl.at[b], flat_v.at[pl.ds(off * 4, n * 4)],
                                   sem)
                  for lvl, n, off in zip(levels, _LVL_N, _LVL_OFF)]
        copies.append(pltpu.async_copy(gtt_hbm.at[b], gtt_v, sem))
        copies.append(pltpu.async_copy(pr_hbm.at[b], pr_v, sem))
        for c in copies:
            c.wait()

        @plsc.parallel_loop(0, _P // _L, unroll=4,
                            carry=jnp.zeros((_L,), jnp.float32))
        def step(i, acc):
            base = i * (2 * _L)
            di = plsc.load_gather(pr_v, [base + evens])
            gi = plsc.load_gather(pr_v, [base + odds])
            d2 = plsc.load_gather(dbt_v, [cols[2], di])
            d3 = plsc.load_gather(dbt_v, [cols[3], di])
            t0 = (plsc.load_gather(gtt_v, [cols[0], gi])
                  - plsc.load_gather(dbt_v, [cols[0], di])) * jnp.exp(-d3)
            t1 = (plsc.load_gather(gtt_v, [cols[1], gi])
                  - plsc.load_gather(dbt_v, [cols[1], di])) * jnp.exp(-d2)
            t2 = plsc.load_gather(gtt_v, [cols[2], gi]) - d2
            t3 = plsc.load_gather(gtt_v, [cols[3], gi]) - d3
            di4 = di * 4
            for c, t in enumerate((t0, t1, t2, t3)):
                p = plsc.load_gather(flat_v, [di4 + cols[c]])
                e = jnp.abs(p - t)
                acc = acc + jnp.where(e < 1.0, (0.5 * e) * e, e - 0.5)
            return acc

        return step

    acc = batch_loss(wid * _BPW)
    for k in range(1, _BPW):
        acc = acc + batch_loss(wid * _BPW + k)
    out_v[...] = acc
    pltpu.sync_copy(out_v, out_hbm.at[wid])


def kernel(boxes0, boxes1, boxes2, boxes3, boxes4, boxes5, gt_boxes, pairs,
           default_boxes):
    levels = [b.reshape(_B, -1)
              for b in (boxes0, boxes1, boxes2, boxes3, boxes4, boxes5)]

    dbt, gtt = pl.pallas_call(
        _prep_body,
        out_shape=[jax.ShapeDtypeStruct((4, _NDB), jnp.float32),
                   jax.ShapeDtypeStruct((_B, 4, _G), jnp.float32)],
    )(default_boxes.T, gt_boxes.transpose(0, 2, 1))

    sc = pl.kernel(
        _sc_body,
        out_type=jax.ShapeDtypeStruct((_NW, _L), jnp.float32),
        mesh=plsc.VectorSubcoreMesh(core_axis_name="c", subcore_axis_name="s"),
        compiler_params=pltpu.CompilerParams(needs_layout_passes=False,
                                             use_tc_tiling_on_sc=False),
        scratch_types=[
            pltpu.VMEM((_NDB * 4,), jnp.float32),
            pltpu.VMEM((4, _NDB), jnp.float32),
            pltpu.VMEM((4, _G), jnp.float32),
            pltpu.VMEM((2 * _P,), jnp.int32),
            pltpu.VMEM((_L,), jnp.float32),
            pltpu.SemaphoreType.DMA,
        ],
    )
    part = sc(*levels, gtt, dbt, pairs.reshape(_B, 2 * _P))
    return jnp.sum(part) / jnp.float32(_B * _P * 4)


# speedup vs baseline: 1.0410x; 1.0410x over previous
import jax
import jax.numpy as jnp
from jax import lax
from jax.experimental import pallas as pl
from jax.experimental.pallas import tpu as pltpu
from jax.experimental.pallas import tpu_sc as plsc

_NDB = 8432
_B, _P, _G = 64, 8192, 256
_LVL_N = (5476, 2166, 600, 150, 36, 4)
_LVL_OFF = (0, 5476, 7642, 8242, 8392, 8428)
_NC, _NS, _L = 2, 16, 16
_NW = _NC * _NS
_BPW = _B // _NW


def _pack(a, b):
    ua = lax.bitcast_convert_type(a.astype(jnp.bfloat16),
                                  jnp.uint16).astype(jnp.uint32)
    ub = lax.bitcast_convert_type(b.astype(jnp.bfloat16),
                                  jnp.uint16).astype(jnp.uint32)
    return lax.bitcast_convert_type(ua | (ub << 16), jnp.int32)


def _prep_body(db_ref, gt_ref, dbp_ref, gtp_ref):
    db = db_ref[...]
    dbp_ref[pl.ds(0, _NDB)] = _pack(db[0, :], db[1, :])
    dbp_ref[pl.ds(_NDB, _NDB)] = _pack(jnp.log(db[2, :]), jnp.log(db[3, :]))
    gt = gt_ref[...]
    gtp_ref[:, pl.ds(0, _G)] = _pack(gt[:, 0, :], gt[:, 1, :])
    gtp_ref[:, pl.ds(_G, _G)] = _pack(jnp.log(gt[:, 2, :]),
                                      jnp.log(gt[:, 3, :]))


def _lo(x):
    return plsc.bitcast(lax.shift_left(x, 16), jnp.float32)


def _hi(x):
    return plsc.bitcast(jnp.bitwise_and(x, jnp.int32(-65536)), jnp.float32)


def _sc_body(l0, l1, l2, l3, l4, l5, gtp_hbm, dbp_hbm, pr_hbm, out_hbm,
             flat_v, dbp_v, gtp_v, pr_v, out_v, sem0, sem1):
    wid = lax.axis_index("c") * _NS + lax.axis_index("s")
    levels = (l0, l1, l2, l3, l4, l5)
    cols = [jnp.full((_L,), c, jnp.int32) for c in range(4)]
    evens = lax.iota(jnp.int32, _L) * 2
    odds = evens + 1
    ndb_c = jnp.full((_L,), _NDB, jnp.int32)
    g_c = jnp.full((_L,), _G, jnp.int32)

    def stage(b, k, sem):
        copies = [pltpu.async_copy(lvl.at[b],
                                   flat_v[k].at[pl.ds(off * 4, n * 4)], sem)
                  for lvl, n, off in zip(levels, _LVL_N, _LVL_OFF)]
        copies.append(pltpu.async_copy(gtp_hbm.at[b], gtp_v[k], sem))
        copies.append(pltpu.async_copy(pr_hbm.at[b], pr_v[k], sem))
        return copies

    def batch_loss(k):
        flat_k, gtp_k, pr_k = flat_v[k], gtp_v[k], pr_v[k]

        @plsc.parallel_loop(0, _P // _L, unroll=4,
                            carry=jnp.zeros((_L,), jnp.float32))
        def step(i, acc):
            base = i * (2 * _L)
            di = plsc.load_gather(pr_k, [base + evens])
            gi = plsc.load_gather(pr_k, [base + odds])
            d01 = plsc.load_gather(dbp_v, [di])
            ld23 = plsc.load_gather(dbp_v, [di + ndb_c])
            g01 = plsc.load_gather(gtp_k, [gi])
            lg23 = plsc.load_gather(gtp_k, [gi + g_c])
            ld2, ld3 = _lo(ld23), _hi(ld23)
            t0 = (_lo(g01) - _lo(d01)) * jnp.exp(-ld3)
            t1 = (_hi(g01) - _hi(d01)) * jnp.exp(-ld2)
            t2 = _lo(lg23) - ld2
            t3 = _hi(lg23) - ld3
            di4 = di * 4
            for c, t in enumerate((t0, t1, t2, t3)):
                p = plsc.load_gather(flat_k, [di4 + cols[c]])
                e = jnp.abs(p - t)
                acc = acc + jnp.where(e < 1.0, (0.5 * e) * e, e - 0.5)
            return acc

        return step

    c_db = pltpu.async_copy(dbp_hbm, dbp_v, sem0)
    cp0 = stage(wid * _BPW, 0, sem0)
    cp1 = stage(wid * _BPW + 1, 1, sem1)
    c_db.wait()
    for c in cp0:
        c.wait()
    acc = batch_loss(0)
    for c in cp1:
        c.wait()
    acc = acc + batch_loss(1)
    out_v[...] = acc
    pltpu.sync_copy(out_v, out_hbm.at[wid])


def kernel(boxes0, boxes1, boxes2, boxes3, boxes4, boxes5, gt_boxes, pairs,
           default_boxes):
    levels = [b.reshape(_B, -1)
              for b in (boxes0, boxes1, boxes2, boxes3, boxes4, boxes5)]

    dbp, gtp = pl.pallas_call(
        _prep_body,
        out_shape=[jax.ShapeDtypeStruct((2 * _NDB,), jnp.int32),
                   jax.ShapeDtypeStruct((_B, 2 * _G), jnp.int32)],
    )(default_boxes.T, gt_boxes.transpose(0, 2, 1))

    sc = pl.kernel(
        _sc_body,
        out_type=jax.ShapeDtypeStruct((_NW, _L), jnp.float32),
        mesh=plsc.VectorSubcoreMesh(core_axis_name="c", subcore_axis_name="s"),
        compiler_params=pltpu.CompilerParams(needs_layout_passes=False,
                                             use_tc_tiling_on_sc=False),
        scratch_types=[
            [pltpu.VMEM((_NDB * 4,), jnp.float32)] * 2,
            pltpu.VMEM((2 * _NDB,), jnp.int32),
            [pltpu.VMEM((2 * _G,), jnp.int32)] * 2,
            [pltpu.VMEM((2 * _P,), jnp.int32)] * 2,
            pltpu.VMEM((_L,), jnp.float32),
            pltpu.SemaphoreType.DMA,
            pltpu.SemaphoreType.DMA,
        ],
    )
    part = sc(*levels, gtp, dbp, pairs.reshape(_B, 2 * _P))
    return jnp.sum(part) / jnp.float32(_B * _P * 4)


# speedup vs baseline: 1.1297x; 1.0852x over previous
import jax
import jax.numpy as jnp
from jax import lax
from jax.experimental import pallas as pl
from jax.experimental.pallas import tpu as pltpu
from jax.experimental.pallas import tpu_sc as plsc

_NDB = 8432
_B, _P, _G = 64, 8192, 256
_LVL_N = (5476, 2166, 600, 150, 36, 4)
_LVL_OFF = (0, 5476, 7642, 8242, 8392, 8428)
_NC, _NS, _L = 2, 16, 16
_NW = _NC * _NS
_BPW = _B // _NW


def _pack(a, b):
    ua = lax.bitcast_convert_type(a.astype(jnp.bfloat16),
                                  jnp.uint16).astype(jnp.uint32)
    ub = lax.bitcast_convert_type(b.astype(jnp.bfloat16),
                                  jnp.uint16).astype(jnp.uint32)
    return lax.bitcast_convert_type(ua | (ub << 16), jnp.int32)


def _prep_body(db_ref, gt_ref, dbp_ref, gtp_ref):
    db = db_ref[...]
    dbp_ref[pl.ds(0, _NDB)] = _pack(db[0, :], db[1, :])
    dbp_ref[pl.ds(_NDB, _NDB)] = _pack(jnp.log(db[2, :]), jnp.log(db[3, :]))
    gt = gt_ref[...]
    gtp_ref[:, pl.ds(0, _G)] = _pack(gt[:, 0, :], gt[:, 1, :])
    gtp_ref[:, pl.ds(_G, _G)] = _pack(jnp.log(gt[:, 2, :]),
                                      jnp.log(gt[:, 3, :]))


def _lo(x):
    return plsc.bitcast(lax.shift_left(x, 16), jnp.float32)


def _hi(x):
    return plsc.bitcast(jnp.bitwise_and(x, jnp.int32(-65536)), jnp.float32)


def _sc_body(l0, l1, l2, l3, l4, l5, gtp_hbm, dbp_hbm, pr_hbm, out_hbm,
             flat_v, dbp_v, gtp_v, di_v, gi_v, out_v, sem0, sem1):
    wid = lax.axis_index("c") * _NS + lax.axis_index("s")
    levels = (l0, l1, l2, l3, l4, l5)
    cols = [jnp.full((_L,), c, jnp.int32) for c in range(4)]
    ndb_c = jnp.full((_L,), _NDB, jnp.int32)
    g_c = jnp.full((_L,), _G, jnp.int32)

    def stage(b, k, sem):
        copies = [pltpu.async_copy(lvl.at[b],
                                   flat_v[k].at[pl.ds(off * 4, n * 4)], sem)
                  for lvl, n, off in zip(levels, _LVL_N, _LVL_OFF)]
        copies.append(pltpu.async_copy(gtp_hbm.at[b], gtp_v[k], sem))
        copies.append(pltpu.async_copy(pr_hbm.at[b, 0], di_v[k], sem))
        copies.append(pltpu.async_copy(pr_hbm.at[b, 1], gi_v[k], sem))
        return copies

    def batch_loss(k):
        flat_k, gtp_k = flat_v[k], gtp_v[k]
        di_k, gi_k = di_v[k], gi_v[k]

        @plsc.parallel_loop(0, _P // _L, unroll=8,
                            carry=jnp.zeros((_L,), jnp.float32))
        def step(i, acc):
            di = di_k[pl.ds(i * _L, _L)]
            gi = gi_k[pl.ds(i * _L, _L)]
            d01 = plsc.load_gather(dbp_v, [di])
            ld23 = plsc.load_gather(dbp_v, [di + ndb_c])
            g01 = plsc.load_gather(gtp_k, [gi])
            lg23 = plsc.load_gather(gtp_k, [gi + g_c])
            ld2, ld3 = _lo(ld23), _hi(ld23)
            t0 = (_lo(g01) - _lo(d01)) * jnp.exp(-ld3)
            t1 = (_hi(g01) - _hi(d01)) * jnp.exp(-ld2)
            t2 = _lo(lg23) - ld2
            t3 = _hi(lg23) - ld3
            di4 = di * 4
            for c, t in enumerate((t0, t1, t2, t3)):
                p = plsc.load_gather(flat_k, [di4 + cols[c]])
                e = jnp.abs(p - t)
                acc = acc + jnp.where(e < 1.0, (0.5 * e) * e, e - 0.5)
            return acc

        return step

    c_db = pltpu.async_copy(dbp_hbm, dbp_v, sem0)
    cp0 = stage(wid * _BPW, 0, sem0)
    cp1 = stage(wid * _BPW + 1, 1, sem1)
    c_db.wait()
    for c in cp0:
        c.wait()
    acc = batch_loss(0)
    for c in cp1:
        c.wait()
    acc = acc + batch_loss(1)
    out_v[...] = acc
    pltpu.sync_copy(out_v, out_hbm.at[wid])


def kernel(boxes0, boxes1, boxes2, boxes3, boxes4, boxes5, gt_boxes, pairs,
           default_boxes):
    levels = [b.reshape(_B, -1)
              for b in (boxes0, boxes1, boxes2, boxes3, boxes4, boxes5)]

    dbp, gtp = pl.pallas_call(
        _prep_body,
        out_shape=[jax.ShapeDtypeStruct((2 * _NDB,), jnp.int32),
                   jax.ShapeDtypeStruct((_B, 2 * _G), jnp.int32)],
    )(default_boxes.T, gt_boxes.transpose(0, 2, 1))

    sc = pl.kernel(
        _sc_body,
        out_type=jax.ShapeDtypeStruct((_NW, _L), jnp.float32),
        mesh=plsc.VectorSubcoreMesh(core_axis_name="c", subcore_axis_name="s"),
        compiler_params=pltpu.CompilerParams(needs_layout_passes=False,
                                             use_tc_tiling_on_sc=False),
        scratch_types=[
            [pltpu.VMEM((_NDB * 4,), jnp.float32)] * 2,
            pltpu.VMEM((2 * _NDB,), jnp.int32),
            [pltpu.VMEM((2 * _G,), jnp.int32)] * 2,
            [pltpu.VMEM((_P,), jnp.int32)] * 2,
            [pltpu.VMEM((_P,), jnp.int32)] * 2,
            pltpu.VMEM((_L,), jnp.float32),
            pltpu.SemaphoreType.DMA,
            pltpu.SemaphoreType.DMA,
        ],
    )
    part = sc(*levels, gtp, dbp, pairs.transpose(0, 2, 1))
    return jnp.sum(part) / jnp.float32(_B * _P * 4)


# speedup vs baseline: 1.2304x; 1.0892x over previous
import jax
import jax.numpy as jnp
from jax import lax
from jax.experimental import pallas as pl
from jax.experimental.pallas import tpu as pltpu
from jax.experimental.pallas import tpu_sc as plsc

_NDB = 8432
_B, _P, _G = 64, 8192, 256
_LVL_N = (5476, 2166, 600, 150, 36, 4)
_LVL_OFF = (0, 5476, 7642, 8242, 8392, 8428)
_NC, _NS, _L = 2, 16, 16
_NW = _NC * _NS
_BPW = _B // _NW


def _pack(a, b):
    ua = lax.bitcast_convert_type(a.astype(jnp.bfloat16),
                                  jnp.uint16).astype(jnp.uint32)
    ub = lax.bitcast_convert_type(b.astype(jnp.bfloat16),
                                  jnp.uint16).astype(jnp.uint32)
    return lax.bitcast_convert_type(ua | (ub << 16), jnp.int32)


def _prep_body(db_ref, gt_ref, dbp_ref, gtp_ref):
    db = db_ref[...]
    dbp_ref[pl.ds(0, _NDB)] = _pack(db[0, :], db[1, :])
    dbp_ref[pl.ds(_NDB, _NDB)] = _pack(jnp.log(db[2, :]), jnp.log(db[3, :]))
    gt = gt_ref[...]
    gtp_ref[:, pl.ds(0, _G)] = _pack(gt[:, 0, :], gt[:, 1, :])
    gtp_ref[:, pl.ds(_G, _G)] = _pack(jnp.log(gt[:, 2, :]),
                                      jnp.log(gt[:, 3, :]))


def _lo(x):
    return plsc.bitcast(lax.shift_left(x, 16), jnp.float32)


def _hi(x):
    return plsc.bitcast(jnp.bitwise_and(x, jnp.int32(-65536)), jnp.float32)


def _sc_body(l0, l1, l2, l3, l4, l5, gtp_hbm, dbp_hbm, pr_hbm, out_hbm,
             flat_v, dbp_v, gtp_v, di_v, gi_v, out_v, sem0, sem1):
    wid = lax.axis_index("c") * _NS + lax.axis_index("s")
    levels = (l0, l1, l2, l3, l4, l5)
    cols = [jnp.full((_L,), c, jnp.int32) for c in range(4)]
    ndb_c = jnp.full((_L,), _NDB, jnp.int32)
    g_c = jnp.full((_L,), _G, jnp.int32)

    def stage(b, k, sem):
        copies = [pltpu.async_copy(lvl.at[b],
                                   flat_v[k].at[pl.ds(off * 4, n * 4)], sem)
                  for lvl, n, off in zip(levels, _LVL_N, _LVL_OFF)]
        copies.append(pltpu.async_copy(gtp_hbm.at[b], gtp_v[k], sem))
        copies.append(pltpu.async_copy(pr_hbm.at[b, 0], di_v[k], sem))
        copies.append(pltpu.async_copy(pr_hbm.at[b, 1], gi_v[k], sem))
        return copies

    def batch_loss(k):
        flat_k, gtp_k = flat_v[k], gtp_v[k]
        di_k, gi_k = di_v[k], gi_v[k]

        @plsc.parallel_loop(0, _P // _L, unroll=4,
                            carry=jnp.zeros((_L,), jnp.float32))
        def step(i, acc):
            di = di_k[pl.ds(i * _L, _L)]
            gi = gi_k[pl.ds(i * _L, _L)]
            d01 = plsc.load_gather(dbp_v, [di])
            ld23 = plsc.load_gather(dbp_v, [di + ndb_c])
            g01 = plsc.load_gather(gtp_k, [gi])
            lg23 = plsc.load_gather(gtp_k, [gi + g_c])
            ld2, ld3 = _lo(ld23), _hi(ld23)
            t0 = (_lo(g01) - _lo(d01)) * jnp.exp(-ld3)
            t1 = (_hi(g01) - _hi(d01)) * jnp.exp(-ld2)
            t2 = _lo(lg23) - ld2
            t3 = _hi(lg23) - ld3
            di4 = di * 4
            for c, t in enumerate((t0, t1, t2, t3)):
                p = plsc.load_gather(flat_k, [di4 + cols[c]])
                e = jnp.abs(p - t)
                acc = acc + jnp.where(e < 1.0, (0.5 * e) * e, e - 0.5)
            return acc

        return step

    c_db = pltpu.async_copy(dbp_hbm, dbp_v, sem0)
    cp0 = stage(wid * _BPW, 0, sem0)
    cp1 = stage(wid * _BPW + 1, 1, sem1)
    c_db.wait()
    for c in cp0:
        c.wait()
    acc = batch_loss(0)
    for c in cp1:
        c.wait()
    acc = acc + batch_loss(1)
    out_v[...] = acc
    pltpu.sync_copy(out_v, out_hbm.at[wid])


def kernel(boxes0, boxes1, boxes2, boxes3, boxes4, boxes5, gt_boxes, pairs,
           default_boxes):
    levels = [b.reshape(_B, -1)
              for b in (boxes0, boxes1, boxes2, boxes3, boxes4, boxes5)]

    dbp, gtp = pl.pallas_call(
        _prep_body,
        out_shape=[jax.ShapeDtypeStruct((2 * _NDB,), jnp.int32),
                   jax.ShapeDtypeStruct((_B, 2 * _G), jnp.int32)],
    )(default_boxes.T, gt_boxes.transpose(0, 2, 1))

    sc = pl.kernel(
        _sc_body,
        out_type=jax.ShapeDtypeStruct((_NW, _L), jnp.float32),
        mesh=plsc.VectorSubcoreMesh(core_axis_name="c", subcore_axis_name="s"),
        compiler_params=pltpu.CompilerParams(needs_layout_passes=False,
                                             use_tc_tiling_on_sc=False),
        scratch_types=[
            [pltpu.VMEM((_NDB * 4,), jnp.float32)] * 2,
            pltpu.VMEM((2 * _NDB,), jnp.int32),
            [pltpu.VMEM((2 * _G,), jnp.int32)] * 2,
            [pltpu.VMEM((_P,), jnp.int32)] * 2,
            [pltpu.VMEM((_P,), jnp.int32)] * 2,
            pltpu.VMEM((_L,), jnp.float32),
            pltpu.SemaphoreType.DMA,
            pltpu.SemaphoreType.DMA,
        ],
    )
    part = sc(*levels, gtp, dbp, pairs.transpose(0, 2, 1))
    return jnp.sum(part) / jnp.float32(_B * _P * 4)
